# Initial kernel scaffold; baseline (speedup 1.0000x reference)
import jax
import jax.numpy as jnp
from jax.experimental import pallas as pl


def kernel(query, key, value, W_out, b_out):
    raise NotImplementedError("write your pallas kernel here")


# speedup vs baseline: 1.6812x; 1.6812x over previous
import functools

import jax
import jax.numpy as jnp
import numpy as np
from jax.experimental import pallas as pl
from jax.experimental.pallas import tpu as pltpu

H = 16
D = 64
E = H * D
BQ = 512


def _fused_attn_kernel(q_ref, k_ref, v_ref, w_ref, b_ref, o_ref):
    scale = 1.0 / np.sqrt(D)
    q = q_ref[0]
    k = k_ref[0]
    v = v_ref[0]
    acc = jnp.broadcast_to(b_ref[...], (q.shape[0], D)).astype(jnp.float32)
    for h in range(H):
        sl = slice(h * D, (h + 1) * D)
        qh = q[:, sl]
        kh = k[:, sl]
        vh = v[:, sl]
        s = jax.lax.dot_general(
            qh, kh, (((1,), (1,)), ((), ())),
            preferred_element_type=jnp.float32) * scale
        s = s - jnp.max(s, axis=-1, keepdims=True)
        e = jnp.exp(s)
        p = e / jnp.sum(e, axis=-1, keepdims=True)
        oh = jax.lax.dot_general(
            p, vh, (((1,), (0,)), ((), ())),
            preferred_element_type=jnp.float32)
        wh = w_ref[:, sl]
        acc = acc + jax.lax.dot_general(
            oh, wh, (((1,), (1,)), ((), ())),
            preferred_element_type=jnp.float32)
    o_ref[0] = acc


@jax.jit
def kernel(query, key, value, W_out, b_out):
    b, n, e = query.shape
    grid = (b, n // BQ)
    out = pl.pallas_call(
        _fused_attn_kernel,
        grid=grid,
        in_specs=[
            pl.BlockSpec((1, BQ, e), lambda bi, qi: (bi, qi, 0)),
            pl.BlockSpec((1, n, e), lambda bi, qi: (bi, 0, 0)),
            pl.BlockSpec((1, n, e), lambda bi, qi: (bi, 0, 0)),
            pl.BlockSpec((D, e), lambda bi, qi: (0, 0)),
            pl.BlockSpec((1, D), lambda bi, qi: (0, 0)),
        ],
        out_specs=pl.BlockSpec((1, BQ, D), lambda bi, qi: (bi, qi, 0)),
        out_shape=jax.ShapeDtypeStruct((b, n, D), jnp.float32),
        compiler_params=pltpu.CompilerParams(
            dimension_semantics=("parallel", "arbitrary"),
        ),
    )(query, key, value, W_out, b_out.reshape(1, D))
    return out


# speedup vs baseline: 1.8554x; 1.1036x over previous
import functools

import jax
import jax.numpy as jnp
import numpy as np
from jax.experimental import pallas as pl
from jax.experimental.pallas import tpu as pltpu

H = 16
D = 64
E = H * D
BQ = 512


def _fused_attn_kernel(q_ref, k_ref, v_ref, w_ref, b_ref, o_ref):
    scale = 1.0 / np.sqrt(D)
    q = q_ref[0]
    k = k_ref[0]
    v = v_ref[0]
    acc = jnp.broadcast_to(b_ref[...], (q.shape[0], D)).astype(jnp.float32)
    for h in range(H):
        sl = slice(h * D, (h + 1) * D)
        qh = q[:, sl].astype(jnp.bfloat16)
        kh = k[:, sl].astype(jnp.bfloat16)
        vh = v[:, sl].astype(jnp.bfloat16)
        s = jax.lax.dot_general(
            qh, kh, (((1,), (1,)), ((), ())),
            preferred_element_type=jnp.float32) * scale
        s = s - jnp.max(s, axis=-1, keepdims=True)
        e = jnp.exp(s)
        denom = jnp.sum(e, axis=-1, keepdims=True)
        oh = jax.lax.dot_general(
            e.astype(jnp.bfloat16), vh, (((1,), (0,)), ((), ())),
            preferred_element_type=jnp.float32) / denom
        wh = w_ref[:, sl]
        acc = acc + jax.lax.dot_general(
            oh, wh, (((1,), (1,)), ((), ())),
            preferred_element_type=jnp.float32)
    o_ref[0] = acc


@jax.jit
def kernel(query, key, value, W_out, b_out):
    b, n, e = query.shape
    grid = (b, n // BQ)
    out = pl.pallas_call(
        _fused_attn_kernel,
        grid=grid,
        in_specs=[
            pl.BlockSpec((1, BQ, e), lambda bi, qi: (bi, qi, 0)),
            pl.BlockSpec((1, n, e), lambda bi, qi: (bi, 0, 0)),
            pl.BlockSpec((1, n, e), lambda bi, qi: (bi, 0, 0)),
            pl.BlockSpec((D, e), lambda bi, qi: (0, 0)),
            pl.BlockSpec((1, D), lambda bi, qi: (0, 0)),
        ],
        out_specs=pl.BlockSpec((1, BQ, D), lambda bi, qi: (bi, qi, 0)),
        out_shape=jax.ShapeDtypeStruct((b, n, D), jnp.float32),
        compiler_params=pltpu.CompilerParams(
            dimension_semantics=("parallel", "arbitrary"),
        ),
    )(query, key, value, W_out, b_out.reshape(1, D))
    return out


# speedup vs baseline: 2.1124x; 1.1385x over previous
import jax
import jax.numpy as jnp
import numpy as np
from jax.experimental import pallas as pl
from jax.experimental.pallas import tpu as pltpu

H = 16
D = 64
E = H * D
BQ = 512


def _fused_attn_kernel(q_ref, k_ref, v_ref, w_ref, b_ref, o_ref):
    q = q_ref[0]
    k = k_ref[0]
    v = v_ref[0]
    acc = jnp.broadcast_to(b_ref[...], (q.shape[0], D)).astype(jnp.float32)
    for h in range(H):
        sl = slice(h * D, (h + 1) * D)
        qh = q[:, sl]
        kh = k[:, sl]
        vh = v[:, sl]
        qf = qh.astype(jnp.float32)
        kf = kh.astype(jnp.float32)
        qn = jnp.sqrt(jnp.sum(qf * qf, axis=1, keepdims=True))
        kn = jnp.sqrt(jnp.max(jnp.sum(kf * kf, axis=1)))
        shift = qn * (kn * 1.01)
        s = jax.lax.dot_general(
            qh, kh, (((1,), (1,)), ((), ())),
            preferred_element_type=jnp.float32)
        e = jnp.exp(s - shift)
        denom = jnp.sum(e, axis=-1, keepdims=True)
        oh = jax.lax.dot_general(
            e.astype(jnp.bfloat16), vh, (((1,), (0,)), ((), ())),
            preferred_element_type=jnp.float32) / denom
        wh = w_ref[:, sl]
        acc = acc + jax.lax.dot_general(
            oh, wh, (((1,), (1,)), ((), ())),
            preferred_element_type=jnp.float32)
    o_ref[0] = acc


@jax.jit
def kernel(query, key, value, W_out, b_out):
    b, n, e = query.shape
    scale = 1.0 / np.sqrt(D)
    qb = (query * scale).astype(jnp.bfloat16)
    kb = key.astype(jnp.bfloat16)
    vb = value.astype(jnp.bfloat16)
    grid = (b, n // BQ)
    out = pl.pallas_call(
        _fused_attn_kernel,
        grid=grid,
        in_specs=[
            pl.BlockSpec((1, BQ, e), lambda bi, qi: (bi, qi, 0)),
            pl.BlockSpec((1, n, e), lambda bi, qi: (bi, 0, 0)),
            pl.BlockSpec((1, n, e), lambda bi, qi: (bi, 0, 0)),
            pl.BlockSpec((D, e), lambda bi, qi: (0, 0)),
            pl.BlockSpec((1, D), lambda bi, qi: (0, 0)),
        ],
        out_specs=pl.BlockSpec((1, BQ, D), lambda bi, qi: (bi, qi, 0)),
        out_shape=jax.ShapeDtypeStruct((b, n, D), jnp.float32),
        compiler_params=pltpu.CompilerParams(
            dimension_semantics=("parallel", "arbitrary"),
        ),
    )(qb, kb, vb, W_out, b_out.reshape(1, D))
    return out


# speedup vs baseline: 2.2761x; 1.0775x over previous
import jax
import jax.numpy as jnp
import numpy as np
from jax.experimental import pallas as pl
from jax.experimental.pallas import tpu as pltpu

H = 16
D = 64
E = H * D
BQ = 512


def _fused_attn_kernel(q_ref, k_ref, v_ref, w_ref, b_ref, o_ref):
    q = q_ref[0]
    k = k_ref[0]
    v = v_ref[0]
    acc = jnp.broadcast_to(b_ref[...], (q.shape[0], D)).astype(jnp.float32)
    for h in range(H):
        sl = slice(h * D, (h + 1) * D)
        qh = q[:, sl]
        kh = k[:, sl]
        vh = v[:, sl]
        qf = qh.astype(jnp.float32)
        kf = kh.astype(jnp.float32)
        qn = jnp.sqrt(jnp.sum(qf * qf, axis=1, keepdims=True))
        kn = jnp.sqrt(jnp.max(jnp.sum(kf * kf, axis=1)))
        shift = qn * (kn * 1.01)
        s = jax.lax.dot_general(
            qh, kh, (((1,), (1,)), ((), ())),
            preferred_element_type=jnp.float32)
        e = jnp.exp(s - shift).astype(jnp.bfloat16)
        vaug = jnp.concatenate(
            [vh, jnp.ones((vh.shape[0], D), jnp.bfloat16)], axis=1)
        oh_full = jax.lax.dot_general(
            e, vaug, (((1,), (0,)), ((), ())),
            preferred_element_type=jnp.float32)
        oh = oh_full[:, :D] / oh_full[:, D:D + 1]
        wh = w_ref[:, sl]
        acc = acc + jax.lax.dot_general(
            oh, wh, (((1,), (1,)), ((), ())),
            preferred_element_type=jnp.float32)
    o_ref[0] = acc


@jax.jit
def kernel(query, key, value, W_out, b_out):
    b, n, e = query.shape
    scale = 1.0 / np.sqrt(D)
    qb = (query * scale).astype(jnp.bfloat16)
    kb = key.astype(jnp.bfloat16)
    vb = value.astype(jnp.bfloat16)
    grid = (b, n // BQ)
    out = pl.pallas_call(
        _fused_attn_kernel,
        grid=grid,
        in_specs=[
            pl.BlockSpec((1, BQ, e), lambda bi, qi: (bi, qi, 0)),
            pl.BlockSpec((1, n, e), lambda bi, qi: (bi, 0, 0)),
            pl.BlockSpec((1, n, e), lambda bi, qi: (bi, 0, 0)),
            pl.BlockSpec((D, e), lambda bi, qi: (0, 0)),
            pl.BlockSpec((1, D), lambda bi, qi: (0, 0)),
        ],
        out_specs=pl.BlockSpec((1, BQ, D), lambda bi, qi: (bi, qi, 0)),
        out_shape=jax.ShapeDtypeStruct((b, n, D), jnp.float32),
        compiler_params=pltpu.CompilerParams(
            dimension_semantics=("parallel", "arbitrary"),
        ),
    )(qb, kb, vb, W_out, b_out.reshape(1, D))
    return out


# speedup vs baseline: 2.3153x; 1.0172x over previous
import jax
import jax.numpy as jnp
import numpy as np
from jax.experimental import pallas as pl
from jax.experimental.pallas import tpu as pltpu

H = 16
D = 64
E = H * D
BQ = 512


def _fused_attn_kernel(q_ref, k_ref, v_ref, w_ref, b_ref, o_ref):
    q = q_ref[0]
    k = k_ref[0]
    v = v_ref[0]
    acc = jnp.broadcast_to(b_ref[...], (q.shape[0], D)).astype(jnp.float32)
    for h in range(H):
        sl = slice(h * D, (h + 1) * D)
        qh = q[:, sl]
        kh = k[:, sl]
        vh = v[:, sl]
        qf = qh.astype(jnp.float32)
        kf = kh.astype(jnp.float32)
        qn = jnp.sqrt(jnp.sum(qf * qf, axis=1, keepdims=True))
        kn = jnp.sqrt(jnp.max(jnp.sum(kf * kf, axis=1)))
        shift = qn * (kn * 1.01)
        s = jax.lax.dot_general(
            qh, kh, (((1,), (1,)), ((), ())),
            preferred_element_type=jnp.float32)
        e = jnp.exp2(s - shift).astype(jnp.bfloat16)
        vaug = jnp.concatenate(
            [vh, jnp.ones((vh.shape[0], D), jnp.bfloat16)], axis=1)
        oh_full = jax.lax.dot_general(
            e, vaug, (((1,), (0,)), ((), ())),
            preferred_element_type=jnp.float32)
        oh = oh_full[:, :D] / oh_full[:, D:D + 1]
        wh = w_ref[:, sl]
        acc = acc + jax.lax.dot_general(
            oh, wh, (((1,), (1,)), ((), ())),
            preferred_element_type=jnp.float32)
    o_ref[0] = acc


@jax.jit
def kernel(query, key, value, W_out, b_out):
    b, n, e = query.shape
    scale = np.log2(np.e) / np.sqrt(D)
    qb = (query * scale).astype(jnp.bfloat16)
    kb = key.astype(jnp.bfloat16)
    vb = value.astype(jnp.bfloat16)
    grid = (b, n // BQ)
    out = pl.pallas_call(
        _fused_attn_kernel,
        grid=grid,
        in_specs=[
            pl.BlockSpec((1, BQ, e), lambda bi, qi: (bi, qi, 0)),
            pl.BlockSpec((1, n, e), lambda bi, qi: (bi, 0, 0)),
            pl.BlockSpec((1, n, e), lambda bi, qi: (bi, 0, 0)),
            pl.BlockSpec((D, e), lambda bi, qi: (0, 0)),
            pl.BlockSpec((1, D), lambda bi, qi: (0, 0)),
        ],
        out_specs=pl.BlockSpec((1, BQ, D), lambda bi, qi: (bi, qi, 0)),
        out_shape=jax.ShapeDtypeStruct((b, n, D), jnp.float32),
        compiler_params=pltpu.CompilerParams(
            dimension_semantics=("parallel", "parallel"),
        ),
    )(qb, kb, vb, W_out, b_out.reshape(1, D))
    return out
